# Initial kernel scaffold; baseline (speedup 1.0000x reference)
import jax
import jax.numpy as jnp
from jax.experimental import pallas as pl


def kernel(x, token_table, pos_table):
    raise NotImplementedError("write your pallas kernel here")


# speedup vs baseline: 1.2715x; 1.2715x over previous
import functools
import jax
import jax.numpy as jnp
from jax import lax
from jax.experimental import pallas as pl
from jax.experimental.pallas import tpu as pltpu
from jax.experimental.pallas import tpu_sc as plsc

BATCH = 4096
SEQ = 200
DIM = 32
NW = 32
BPW = BATCH // NW


def _body(x_hbm, tok_hbm, pos_hbm, out_hbm, idx_v, pos_v, buf_v, gsem):
  cid = lax.axis_index("c")
  sid = lax.axis_index("s")
  wid = sid * 2 + cid

  pltpu.sync_copy(x_hbm.at[:, wid], idx_v)
  pltpu.sync_copy(pos_hbm, pos_v)

  @pl.loop(0, SEQ)
  def _l_loop(l):
    pltpu.async_copy(tok_hbm.at[idx_v.at[l]], buf_v, gsem).wait()

    p0 = pos_v[l, 0:16]
    p1 = pos_v[l, 16:32]

    @pl.loop(0, BPW, unroll=8)
    def _r_loop(r):
      buf_v[r, 0:16] += p0
      buf_v[r, 16:32] += p1

    pltpu.sync_copy(buf_v, out_hbm.at[pl.ds(wid * BPW, BPW), l])


def kernel(x, token_table, pos_table):
  x_t = x.astype(jnp.int32).T.reshape(SEQ, NW, BPW)
  mesh = plsc.VectorSubcoreMesh(core_axis_name="c", subcore_axis_name="s")
  run = pl.kernel(
      _body,
      out_type=jax.ShapeDtypeStruct((BATCH, SEQ, DIM), jnp.float32),
      mesh=mesh,
      compiler_params=pltpu.CompilerParams(use_tc_tiling_on_sc=False),
      scratch_types=[
          pltpu.VMEM((SEQ, BPW), jnp.int32),
          pltpu.VMEM((SEQ, DIM), jnp.float32),
          pltpu.VMEM((BPW, DIM), jnp.float32),
          pltpu.SemaphoreType.DMA,
      ],
  )
  return run(x_t, token_table, pos_table)


# speedup vs baseline: 1.4981x; 1.1782x over previous
import functools
import jax
import jax.numpy as jnp
from jax import lax
from jax.experimental import pallas as pl
from jax.experimental.pallas import tpu as pltpu
from jax.experimental.pallas import tpu_sc as plsc

BATCH = 4096
SEQ = 200
DIM = 32
NW = 32
BPW = BATCH // NW
NBUF = 8
G = 6


def _body(x_hbm, tok_hbm, pos_hbm, out_hbm, idx_v, pos_v, buf_v, gsem, ssem):
  cid = lax.axis_index("c")
  sid = lax.axis_index("s")
  wid = sid * 2 + cid

  pltpu.sync_copy(x_hbm.at[:, wid], idx_v)
  pltpu.sync_copy(pos_hbm, pos_v)

  def start_gather(m, b):
    pltpu.async_copy(tok_hbm.at[idx_v.at[m]], buf_v.at[b], gsem.at[b])

  def wait_gather(m, b):
    pltpu.make_async_copy(tok_hbm.at[idx_v.at[m]], buf_v.at[b],
                          gsem.at[b]).wait()

  def out_slice(l):
    return out_hbm.at[pl.ds(wid * BPW, BPW), l]

  for j in range(G):
    start_gather(j, j)

  @pl.loop(0, SEQ, step=NBUF)
  def _outer(l0):
    for j in range(NBUF):
      l = l0 + j
      b = j
      wait_gather(l, b)

      p0 = pos_v[l, 0:16]
      p1 = pos_v[l, 16:32]

      @pl.loop(0, BPW, unroll=8)
      def _r_loop(r):
        buf_v[b, r, 0:16] += p0
        buf_v[b, r, 16:32] += p1

      pltpu.async_copy(buf_v.at[b], out_slice(l), ssem.at[b])

      m = l + G
      mb = (j + G) % NBUF

      @pl.when(m < SEQ)
      def _():
        @pl.when(m >= NBUF)
        def _():
          pltpu.make_async_copy(buf_v.at[mb], out_slice(m - NBUF),
                                ssem.at[mb]).wait()
        start_gather(m, mb)

  for j in range(NBUF):
    pltpu.make_async_copy(buf_v.at[j], out_slice(SEQ - NBUF + j),
                          ssem.at[j]).wait()


def kernel(x, token_table, pos_table):
  x_t = x.astype(jnp.int32).T.reshape(SEQ, NW, BPW)
  mesh = plsc.VectorSubcoreMesh(core_axis_name="c", subcore_axis_name="s")
  run = pl.kernel(
      _body,
      out_type=jax.ShapeDtypeStruct((BATCH, SEQ, DIM), jnp.float32),
      mesh=mesh,
      compiler_params=pltpu.CompilerParams(use_tc_tiling_on_sc=False),
      scratch_types=[
          pltpu.VMEM((SEQ, BPW), jnp.int32),
          pltpu.VMEM((SEQ, DIM), jnp.float32),
          pltpu.VMEM((NBUF, BPW, DIM), jnp.float32),
          pltpu.SemaphoreType.DMA((NBUF,)),
          pltpu.SemaphoreType.DMA((NBUF,)),
      ],
  )
  return run(x_t, token_table, pos_table)
